# baseline (speedup 1.0000x reference)
import functools

import jax
import jax.numpy as jnp
from jax import lax
from jax.experimental import pallas as pl
from jax.experimental.pallas import tpu as pltpu
from jax.experimental.pallas import tpu_sc as plsc

N_NODES = 10000
N_EDGES = 320000
FEAT = 128

NC = 2
NS = 16
NW = NC * NS
EPT = N_EDGES // NW
FPT = FEAT // NW
NPAD = 10240
G = 2000
NGRP = N_EDGES // G
CPG = G // 16


def _proj_body(x_ref, w_ref, a_ref, h_ref, s_ref):
    h = jnp.dot(x_ref[...], w_ref[...], preferred_element_type=jnp.float32,
                precision=lax.Precision.HIGHEST)
    h_ref[...] = h
    s_ref[...] = jnp.dot(h, a_ref[...], preferred_element_type=jnp.float32,
                         precision=lax.Precision.HIGHEST)


_proj = pl.pallas_call(
    _proj_body,
    grid=(10,),
    in_specs=[
        pl.BlockSpec((N_NODES // 10, FEAT), lambda i: (i, 0)),
        pl.BlockSpec((FEAT, FEAT), lambda i: (0, 0)),
        pl.BlockSpec((FEAT, 2), lambda i: (0, 0)),
    ],
    out_specs=[
        pl.BlockSpec((N_NODES // 10, FEAT), lambda i: (i, 0)),
        pl.BlockSpec((N_NODES // 10, 2), lambda i: (i, 0)),
    ],
    out_shape=[
        jax.ShapeDtypeStruct((N_NODES, FEAT), jnp.float32),
        jax.ShapeDtypeStruct((N_NODES, 2), jnp.float32),
    ],
)


_mesh = plsc.VectorSubcoreMesh(core_axis_name="c", subcore_axis_name="s")


@functools.partial(
    pl.kernel,
    out_type=[
        jax.ShapeDtypeStruct((N_EDGES,), jnp.float32),
        jax.ShapeDtypeStruct((NW, NPAD), jnp.float32),
    ],
    mesh=_mesh,
    compiler_params=pltpu.CompilerParams(needs_layout_passes=False),
    scratch_types=[
        pltpu.VMEM((EPT,), jnp.int32),
        pltpu.VMEM((EPT,), jnp.int32),
        pltpu.VMEM((N_NODES,), jnp.float32),
        pltpu.VMEM((N_NODES,), jnp.float32),
        pltpu.VMEM((EPT,), jnp.float32),
        pltpu.VMEM((NPAD,), jnp.float32),
    ],
)
def _wden_kernel(s1_hbm, s2_hbm, src_hbm, dst_hbm, w_out, den_out,
                 src_v, dst_v, s1_v, s2_v, w_v, den_v):
    c = lax.axis_index("c")
    s = lax.axis_index("s")
    wid = s * NC + c

    pltpu.sync_copy(src_hbm.at[pl.ds(wid * EPT, EPT)], src_v)
    pltpu.sync_copy(dst_hbm.at[pl.ds(wid * EPT, EPT)], dst_v)
    pltpu.sync_copy(s1_hbm, s1_v)
    pltpu.sync_copy(s2_hbm, s2_v)

    zero = jnp.zeros((16,), jnp.float32)

    def _zero_den(i, _):
        den_v[pl.ds(i * 16, 16)] = zero
        return 0

    lax.fori_loop(0, NPAD // 16, _zero_den, 0)

    def _chunk(i, _):
        base = i * 16
        s16 = src_v[pl.ds(base, 16)]
        d16 = dst_v[pl.ds(base, 16)]
        e = plsc.load_gather(s1_v, [s16]) + plsc.load_gather(s2_v, [d16])
        e = jnp.maximum(e, 0.2 * e)
        w = jnp.exp(e)
        w_v[pl.ds(base, 16)] = w
        plsc.addupdate_scatter(den_v, [d16], w)
        return 0

    lax.fori_loop(0, EPT // 16, _chunk, 0)

    pltpu.sync_copy(w_v, w_out.at[pl.ds(wid * EPT, EPT)])
    pltpu.sync_copy(den_v, den_out.at[wid])


@functools.partial(
    pl.kernel,
    out_type=jax.ShapeDtypeStruct((NW, FPT * NPAD), jnp.float32),
    mesh=_mesh,
    compiler_params=pltpu.CompilerParams(needs_layout_passes=False),
    scratch_types=[
        pltpu.VMEM((FPT * N_NODES,), jnp.float32),
        pltpu.VMEM((FPT * NPAD,), jnp.float32),
        [pltpu.VMEM((G,), jnp.int32) for _ in range(2)],
        [pltpu.VMEM((G,), jnp.int32) for _ in range(2)],
        [pltpu.VMEM((G,), jnp.float32) for _ in range(2)],
        [pltpu.SemaphoreType.DMA for _ in range(2)],
    ],
)
def _colagg_kernel(ht_hbm, src_hbm, dst_hbm, w_hbm, acc_out,
                   hcol_v, acc_v, srcg, dstg, wg, sems):
    c = lax.axis_index("c")
    s = lax.axis_index("s")
    wid = s * NC + c

    pltpu.sync_copy(ht_hbm.at[pl.ds(wid * (FPT * N_NODES), FPT * N_NODES)],
                    hcol_v)

    zero = jnp.zeros((16,), jnp.float32)

    def _zero_acc(i, _):
        acc_v[pl.ds(i * 16, 16)] = zero
        return 0

    lax.fori_loop(0, (FPT * NPAD) // 16, _zero_acc, 0)

    def _stage(g, b):
        sl = pl.ds(g * G, G)
        pltpu.async_copy(src_hbm.at[sl], srcg[b], sems[b])
        pltpu.async_copy(dst_hbm.at[sl], dstg[b], sems[b])
        pltpu.async_copy(w_hbm.at[sl], wg[b], sems[b])

    def _drain(b):
        sl = pl.ds(0, G)
        pltpu.make_async_copy(src_hbm.at[sl], srcg[b], sems[b]).wait()
        pltpu.make_async_copy(dst_hbm.at[sl], dstg[b], sems[b]).wait()
        pltpu.make_async_copy(w_hbm.at[sl], wg[b], sems[b]).wait()

    _stage(0, 0)
    _stage(1, 1)

    def _group(g, _):
        for b in range(2):
            gg = g * 2 + b
            _drain(b)
            sv, dv, wv = srcg[b], dstg[b], wg[b]

            def _chunk(i, _):
                for cc in range(5):
                    base = (i * 5 + cc) * 16
                    s16 = sv[pl.ds(base, 16)]
                    d16 = dv[pl.ds(base, 16)]
                    w16 = wv[pl.ds(base, 16)]
                    for fl in range(FPT):
                        col = plsc.load_gather(hcol_v, [s16 + fl * N_NODES])
                        plsc.addupdate_scatter(acc_v, [d16 + fl * NPAD],
                                               col * w16)
                return 0

            lax.fori_loop(0, CPG // 5, _chunk, 0)

            @pl.when(gg + 2 < NGRP)
            def _():
                _stage(gg + 2, b)
        return 0

    lax.fori_loop(0, NGRP // 2, _group, 0)

    pltpu.sync_copy(acc_v, acc_out.at[wid])


def _combine_body(acc_ref, den_ref, o_ref):
    a = acc_ref[...]
    den = jnp.sum(den_ref[...], axis=0)
    den = jnp.where(den == 0.0, 1.0, den)
    x = a / den[None, :]
    o_ref[...] = jnp.where(x > 0, x, jnp.exp(jnp.minimum(x, 0.0)) - 1.0)


_CBLK = 1024

_combine = pl.pallas_call(
    _combine_body,
    grid=(NPAD // _CBLK,),
    in_specs=[
        pl.BlockSpec((FEAT, _CBLK), lambda i: (0, i)),
        pl.BlockSpec((NW, _CBLK), lambda i: (0, i)),
    ],
    out_specs=pl.BlockSpec((FEAT, _CBLK), lambda i: (0, i)),
    out_shape=jax.ShapeDtypeStruct((FEAT, NPAD), jnp.float32),
)


def kernel(features, edge_src, edge_dst, W, a):
    n, b, f = features.shape
    x = features.reshape(n, f)
    A = a.reshape(2, f).T
    h, sc = _proj(x, W, A)
    w_e, den = _wden_kernel(sc[:, 0], sc[:, 1], edge_src, edge_dst)
    ht_flat = h.T.reshape(-1)
    acc = _colagg_kernel(ht_flat, edge_src, edge_dst, w_e)
    acc_t = acc.reshape(FEAT, NPAD)
    out_t = _combine(acc_t, den)
    return out_t[:, :n].T.reshape(n, b, f)


# speedup vs baseline: 2.1684x; 2.1684x over previous
import functools

import jax
import jax.numpy as jnp
from jax import lax
from jax.experimental import pallas as pl
from jax.experimental.pallas import tpu as pltpu
from jax.experimental.pallas import tpu_sc as plsc

N_NODES = 10000
N_EDGES = 320000
FEAT = 128

NC = 2
NS = 16
NW = NC * NS
EPT = N_EDGES // NW
FPT = FEAT // NW
NPAD = 10240
G = 2000
NGRP = N_EDGES // G
CPG = G // 16


def _proj_body(x_ref, w_ref, a_ref, h_ref, s_ref):
    h = jnp.dot(x_ref[...], w_ref[...], preferred_element_type=jnp.float32,
                precision=lax.Precision.HIGHEST)
    h_ref[...] = h
    s_ref[...] = jnp.dot(h, a_ref[...], preferred_element_type=jnp.float32,
                         precision=lax.Precision.HIGHEST)


_proj = pl.pallas_call(
    _proj_body,
    grid=(10,),
    in_specs=[
        pl.BlockSpec((N_NODES // 10, FEAT), lambda i: (i, 0)),
        pl.BlockSpec((FEAT, FEAT), lambda i: (0, 0)),
        pl.BlockSpec((FEAT, 2), lambda i: (0, 0)),
    ],
    out_specs=[
        pl.BlockSpec((N_NODES // 10, FEAT), lambda i: (i, 0)),
        pl.BlockSpec((N_NODES // 10, 2), lambda i: (i, 0)),
    ],
    out_shape=[
        jax.ShapeDtypeStruct((N_NODES, FEAT), jnp.float32),
        jax.ShapeDtypeStruct((N_NODES, 2), jnp.float32),
    ],
)


_mesh = plsc.VectorSubcoreMesh(core_axis_name="c", subcore_axis_name="s")


@functools.partial(
    pl.kernel,
    out_type=[
        jax.ShapeDtypeStruct((N_EDGES,), jnp.float32),
        jax.ShapeDtypeStruct((NW, NPAD), jnp.float32),
    ],
    mesh=_mesh,
    compiler_params=pltpu.CompilerParams(needs_layout_passes=False),
    scratch_types=[
        pltpu.VMEM((EPT,), jnp.int32),
        pltpu.VMEM((EPT,), jnp.int32),
        pltpu.VMEM((N_NODES,), jnp.float32),
        pltpu.VMEM((N_NODES,), jnp.float32),
        pltpu.VMEM((EPT,), jnp.float32),
        pltpu.VMEM((NPAD,), jnp.float32),
    ],
)
def _wden_kernel(s1_hbm, s2_hbm, src_hbm, dst_hbm, w_out, den_out,
                 src_v, dst_v, s1_v, s2_v, w_v, den_v):
    c = lax.axis_index("c")
    s = lax.axis_index("s")
    wid = s * NC + c

    pltpu.sync_copy(src_hbm.at[pl.ds(wid * EPT, EPT)], src_v)
    pltpu.sync_copy(dst_hbm.at[pl.ds(wid * EPT, EPT)], dst_v)
    pltpu.sync_copy(s1_hbm, s1_v)
    pltpu.sync_copy(s2_hbm, s2_v)

    zero = jnp.zeros((16,), jnp.float32)

    def _zero_den(i, _):
        den_v[pl.ds(i * 16, 16)] = zero
        return 0

    lax.fori_loop(0, NPAD // 16, _zero_den, 0)

    def _chunk(i, _):
        base = i * 16
        s16 = src_v[pl.ds(base, 16)]
        d16 = dst_v[pl.ds(base, 16)]
        e = plsc.load_gather(s1_v, [s16]) + plsc.load_gather(s2_v, [d16])
        e = jnp.maximum(e, 0.2 * e)
        w = jnp.exp(e)
        w_v[pl.ds(base, 16)] = w
        plsc.addupdate_scatter(den_v, [d16], w)
        return 0

    lax.fori_loop(0, EPT // 16, _chunk, 0)

    pltpu.sync_copy(w_v, w_out.at[pl.ds(wid * EPT, EPT)])
    pltpu.sync_copy(den_v, den_out.at[wid])


@functools.partial(
    pl.kernel,
    out_type=jax.ShapeDtypeStruct((NW, FPT * NPAD), jnp.float32),
    mesh=_mesh,
    compiler_params=pltpu.CompilerParams(needs_layout_passes=False),
    scratch_types=[
        pltpu.VMEM((FPT * N_NODES,), jnp.float32),
        pltpu.VMEM((FPT * NPAD,), jnp.float32),
        [pltpu.VMEM((G,), jnp.int32) for _ in range(2)],
        [pltpu.VMEM((G,), jnp.int32) for _ in range(2)],
        [pltpu.VMEM((G,), jnp.float32) for _ in range(2)],
        [pltpu.SemaphoreType.DMA for _ in range(2)],
    ],
)
def _colagg_kernel(ht_hbm, src_hbm, dst_hbm, w_hbm, acc_out,
                   hcol_v, acc_v, srcg, dstg, wg, sems):
    c = lax.axis_index("c")
    s = lax.axis_index("s")
    wid = s * NC + c

    pltpu.sync_copy(ht_hbm.at[pl.ds(wid * (FPT * N_NODES), FPT * N_NODES)],
                    hcol_v)

    zero = jnp.zeros((16,), jnp.float32)

    def _zero_acc(i, _):
        acc_v[pl.ds(i * 16, 16)] = zero
        return 0

    lax.fori_loop(0, (FPT * NPAD) // 16, _zero_acc, 0)

    def _stage(g, b):
        sl = pl.ds(g * G, G)
        pltpu.async_copy(src_hbm.at[sl], srcg[b], sems[b])
        pltpu.async_copy(dst_hbm.at[sl], dstg[b], sems[b])
        pltpu.async_copy(w_hbm.at[sl], wg[b], sems[b])

    def _drain(b):
        sl = pl.ds(0, G)
        pltpu.make_async_copy(src_hbm.at[sl], srcg[b], sems[b]).wait()
        pltpu.make_async_copy(dst_hbm.at[sl], dstg[b], sems[b]).wait()
        pltpu.make_async_copy(w_hbm.at[sl], wg[b], sems[b]).wait()

    _stage(0, 0)
    _stage(1, 1)

    def _group(g, _):
        for b in range(2):
            gg = g * 2 + b
            _drain(b)
            sv, dv, wv = srcg[b], dstg[b], wg[b]

            @plsc.parallel_loop(0, CPG, 1, unroll=5)
            def _chunk(i):
                base = i * 16
                s16 = sv[pl.ds(base, 16)]
                d16 = dv[pl.ds(base, 16)]
                w16 = wv[pl.ds(base, 16)]
                for fl in range(FPT):
                    col = plsc.load_gather(hcol_v, [s16 + fl * N_NODES])
                    plsc.addupdate_scatter(acc_v, [d16 + fl * NPAD],
                                           col * w16)

            @pl.when(gg + 2 < NGRP)
            def _():
                _stage(gg + 2, b)
        return 0

    lax.fori_loop(0, NGRP // 2, _group, 0)

    pltpu.sync_copy(acc_v, acc_out.at[wid])


def _combine_body(acc_ref, den_ref, o_ref):
    a = acc_ref[...]
    den = jnp.sum(den_ref[...], axis=0)
    den = jnp.where(den == 0.0, 1.0, den)
    x = a / den[None, :]
    o_ref[...] = jnp.where(x > 0, x, jnp.exp(jnp.minimum(x, 0.0)) - 1.0)


_CBLK = 1024

_combine = pl.pallas_call(
    _combine_body,
    grid=(NPAD // _CBLK,),
    in_specs=[
        pl.BlockSpec((FEAT, _CBLK), lambda i: (0, i)),
        pl.BlockSpec((NW, _CBLK), lambda i: (0, i)),
    ],
    out_specs=pl.BlockSpec((FEAT, _CBLK), lambda i: (0, i)),
    out_shape=jax.ShapeDtypeStruct((FEAT, NPAD), jnp.float32),
)


def kernel(features, edge_src, edge_dst, W, a):
    n, b, f = features.shape
    x = features.reshape(n, f)
    A = a.reshape(2, f).T
    h, sc = _proj(x, W, A)
    w_e, den = _wden_kernel(sc[:, 0], sc[:, 1], edge_src, edge_dst)
    ht_flat = h.T.reshape(-1)
    acc = _colagg_kernel(ht_flat, edge_src, edge_dst, w_e)
    acc_t = acc.reshape(FEAT, NPAD)
    out_t = _combine(acc_t, den)
    return out_t[:, :n].T.reshape(n, b, f)


# speedup vs baseline: 2.2373x; 1.0318x over previous
import functools

import jax
import jax.numpy as jnp
from jax import lax
from jax.experimental import pallas as pl
from jax.experimental.pallas import tpu as pltpu
from jax.experimental.pallas import tpu_sc as plsc

N_NODES = 10000
N_EDGES = 320000
FEAT = 128

NC = 2
NS = 16
NW = NC * NS
EPT = N_EDGES // NW
FPT = FEAT // NW
NPAD = 10240
G = 2000
NGRP = N_EDGES // G
CPG = G // 16


def _proj_body(x_ref, w_ref, a_ref, h_ref, s_ref):
    h = jnp.dot(x_ref[...], w_ref[...], preferred_element_type=jnp.float32,
                precision=lax.Precision.HIGHEST)
    h_ref[...] = h
    s_ref[...] = jnp.dot(h, a_ref[...], preferred_element_type=jnp.float32,
                         precision=lax.Precision.HIGHEST)


_proj = pl.pallas_call(
    _proj_body,
    grid=(10,),
    in_specs=[
        pl.BlockSpec((N_NODES // 10, FEAT), lambda i: (i, 0)),
        pl.BlockSpec((FEAT, FEAT), lambda i: (0, 0)),
        pl.BlockSpec((FEAT, 2), lambda i: (0, 0)),
    ],
    out_specs=[
        pl.BlockSpec((N_NODES // 10, FEAT), lambda i: (i, 0)),
        pl.BlockSpec((N_NODES // 10, 2), lambda i: (i, 0)),
    ],
    out_shape=[
        jax.ShapeDtypeStruct((N_NODES, FEAT), jnp.float32),
        jax.ShapeDtypeStruct((N_NODES, 2), jnp.float32),
    ],
)


_mesh = plsc.VectorSubcoreMesh(core_axis_name="c", subcore_axis_name="s")


@functools.partial(
    pl.kernel,
    out_type=[
        jax.ShapeDtypeStruct((N_EDGES,), jnp.float32),
        jax.ShapeDtypeStruct((NW, NPAD), jnp.float32),
    ],
    mesh=_mesh,
    compiler_params=pltpu.CompilerParams(needs_layout_passes=False),
    scratch_types=[
        pltpu.VMEM((EPT,), jnp.int32),
        pltpu.VMEM((EPT,), jnp.int32),
        pltpu.VMEM((N_NODES,), jnp.float32),
        pltpu.VMEM((N_NODES,), jnp.float32),
        pltpu.VMEM((EPT,), jnp.float32),
        pltpu.VMEM((NPAD,), jnp.float32),
    ],
)
def _wden_kernel(s1_hbm, s2_hbm, src_hbm, dst_hbm, w_out, den_out,
                 src_v, dst_v, s1_v, s2_v, w_v, den_v):
    c = lax.axis_index("c")
    s = lax.axis_index("s")
    wid = s * NC + c

    pltpu.sync_copy(src_hbm.at[pl.ds(wid * EPT, EPT)], src_v)
    pltpu.sync_copy(dst_hbm.at[pl.ds(wid * EPT, EPT)], dst_v)
    pltpu.sync_copy(s1_hbm, s1_v)
    pltpu.sync_copy(s2_hbm, s2_v)

    zero = jnp.zeros((16,), jnp.float32)

    def _zero_den(i, _):
        den_v[pl.ds(i * 16, 16)] = zero
        return 0

    lax.fori_loop(0, NPAD // 16, _zero_den, 0)

    @plsc.parallel_loop(0, EPT // 16, 1, unroll=5)
    def _chunk(i):
        base = i * 16
        s16 = src_v[pl.ds(base, 16)]
        d16 = dst_v[pl.ds(base, 16)]
        e = plsc.load_gather(s1_v, [s16]) + plsc.load_gather(s2_v, [d16])
        e = jnp.maximum(e, 0.2 * e)
        w = jnp.exp(e)
        w_v[pl.ds(base, 16)] = w
        plsc.addupdate_scatter(den_v, [d16], w)

    pltpu.sync_copy(w_v, w_out.at[pl.ds(wid * EPT, EPT)])
    pltpu.sync_copy(den_v, den_out.at[wid])


@functools.partial(
    pl.kernel,
    out_type=jax.ShapeDtypeStruct((NW, FPT * NPAD), jnp.float32),
    mesh=_mesh,
    compiler_params=pltpu.CompilerParams(needs_layout_passes=False),
    scratch_types=[
        pltpu.VMEM((FPT * N_NODES,), jnp.float32),
        pltpu.VMEM((FPT * NPAD,), jnp.float32),
        [pltpu.VMEM((G,), jnp.int32) for _ in range(2)],
        [pltpu.VMEM((G,), jnp.int32) for _ in range(2)],
        [pltpu.VMEM((G,), jnp.float32) for _ in range(2)],
        [pltpu.SemaphoreType.DMA for _ in range(2)],
    ],
)
def _colagg_kernel(ht_hbm, src_hbm, dst_hbm, w_hbm, acc_out,
                   hcol_v, acc_v, srcg, dstg, wg, sems):
    c = lax.axis_index("c")
    s = lax.axis_index("s")
    wid = s * NC + c

    pltpu.sync_copy(ht_hbm.at[pl.ds(wid * (FPT * N_NODES), FPT * N_NODES)],
                    hcol_v)

    zero = jnp.zeros((16,), jnp.float32)

    def _zero_acc(i, _):
        acc_v[pl.ds(i * 16, 16)] = zero
        return 0

    lax.fori_loop(0, (FPT * NPAD) // 16, _zero_acc, 0)

    def _stage(g, b):
        sl = pl.ds(g * G, G)
        pltpu.async_copy(src_hbm.at[sl], srcg[b], sems[b])
        pltpu.async_copy(dst_hbm.at[sl], dstg[b], sems[b])
        pltpu.async_copy(w_hbm.at[sl], wg[b], sems[b])

    def _drain(b):
        sl = pl.ds(0, G)
        pltpu.make_async_copy(src_hbm.at[sl], srcg[b], sems[b]).wait()
        pltpu.make_async_copy(dst_hbm.at[sl], dstg[b], sems[b]).wait()
        pltpu.make_async_copy(w_hbm.at[sl], wg[b], sems[b]).wait()

    _stage(0, 0)
    _stage(1, 1)

    def _group(g, _):
        for b in range(2):
            gg = g * 2 + b
            _drain(b)
            sv, dv, wv = srcg[b], dstg[b], wg[b]

            @plsc.parallel_loop(0, CPG, 1, unroll=5)
            def _chunk(i):
                base = i * 16
                s16 = sv[pl.ds(base, 16)]
                d16 = dv[pl.ds(base, 16)]
                w16 = wv[pl.ds(base, 16)]
                for fl in range(FPT):
                    col = plsc.load_gather(hcol_v, [s16 + fl * N_NODES])
                    plsc.addupdate_scatter(acc_v, [d16 + fl * NPAD],
                                           col * w16)

            @pl.when(gg + 2 < NGRP)
            def _():
                _stage(gg + 2, b)
        return 0

    lax.fori_loop(0, NGRP // 2, _group, 0)

    pltpu.sync_copy(acc_v, acc_out.at[wid])


def _combine_body(acc_ref, den_ref, o_ref):
    a = acc_ref[...]
    den = jnp.sum(den_ref[...], axis=0)
    den = jnp.where(den == 0.0, 1.0, den)
    x = a / den[None, :]
    o_ref[...] = jnp.where(x > 0, x, jnp.exp(jnp.minimum(x, 0.0)) - 1.0)


_CBLK = 1024

_combine = pl.pallas_call(
    _combine_body,
    grid=(NPAD // _CBLK,),
    in_specs=[
        pl.BlockSpec((FEAT, _CBLK), lambda i: (0, i)),
        pl.BlockSpec((NW, _CBLK), lambda i: (0, i)),
    ],
    out_specs=pl.BlockSpec((FEAT, _CBLK), lambda i: (0, i)),
    out_shape=jax.ShapeDtypeStruct((FEAT, NPAD), jnp.float32),
)


def kernel(features, edge_src, edge_dst, W, a):
    n, b, f = features.shape
    x = features.reshape(n, f)
    A = a.reshape(2, f).T
    h, sc = _proj(x, W, A)
    w_e, den = _wden_kernel(sc[:, 0], sc[:, 1], edge_src, edge_dst)
    ht_flat = h.T.reshape(-1)
    acc = _colagg_kernel(ht_flat, edge_src, edge_dst, w_e)
    acc_t = acc.reshape(FEAT, NPAD)
    out_t = _combine(acc_t, den)
    return out_t[:, :n].T.reshape(n, b, f)
